# baseline (speedup 1.0000x reference)
import functools
import math

import jax
import jax.numpy as jnp
from jax import lax
from jax.experimental import pallas as pl
from jax.experimental.pallas import tpu as pltpu
from jax.experimental.pallas import tpu_sc as plsc

VOCAB = 1000000
EMB = 64
SCALE = math.sqrt(EMB)

B = 4096 * 200
NW = 32
BPW = B // NW
IW = 128
CH = 512
NCH = BPW // CH
IROWS = CH // IW

_mesh = plsc.VectorSubcoreMesh(core_axis_name="c", subcore_axis_name="s")


@functools.partial(
    pl.kernel,
    mesh=_mesh,
    out_type=jax.ShapeDtypeStruct((B, EMB), jnp.float32),
    scratch_types=[
        pltpu.VMEM((IROWS, IW), jnp.int32),
        pltpu.VMEM((CH, EMB), jnp.float32),
        pltpu.SemaphoreType.DMA,
    ],
    compiler_params=pltpu.CompilerParams(use_tc_tiling_on_sc=False),
)
def _emb_lookup(tokens_hbm, table_hbm, out_hbm, idx_v, rows_v, sem):
    wid = lax.axis_index("s") * 2 + lax.axis_index("c")
    row0 = wid * (BPW // IW)

    def chunk(g, carry):
        pltpu.sync_copy(tokens_hbm.at[pl.ds(row0 + g * IROWS, IROWS)], idx_v)
        for j in range(IROWS):
            pltpu.async_copy(
                table_hbm.at[idx_v.at[j]],
                rows_v.at[pl.ds(j * IW, IW)],
                sem,
            )
        for j in range(IROWS):
            pltpu.make_async_copy(
                table_hbm.at[idx_v.at[j]],
                rows_v.at[pl.ds(j * IW, IW)],
                sem,
            ).wait()

        def scale_row(i, c):
            for jj in range(EMB // 16):
                sl = pl.ds(jj * 16, 16)
                rows_v[i, sl] = rows_v[i, sl] * SCALE
            return c

        lax.fori_loop(0, CH, scale_row, 0)
        pltpu.sync_copy(rows_v, out_hbm.at[pl.ds(wid * BPW + g * CH, CH)])
        return carry

    lax.fori_loop(0, NCH, chunk, 0)


def kernel(tokens, table):
    tokens2d = tokens.astype(jnp.int32).reshape(B // IW, IW)
    out = _emb_lookup(tokens2d, table)
    return out.reshape(tokens.shape[0], tokens.shape[1], EMB)


# speedup vs baseline: 1.1378x; 1.1378x over previous
import functools
import math

import jax
import jax.numpy as jnp
from jax import lax
from jax.experimental import pallas as pl
from jax.experimental.pallas import tpu as pltpu
from jax.experimental.pallas import tpu_sc as plsc

EMB = 64
SCALE = math.sqrt(EMB)

B = 4096 * 200
NW = 32
BPW = B // NW
IW = 128
UNIT = 256
IPU = UNIT // IW
NU = BPW // UNIT
NBUF = 4
NROUND = NU // NBUF
IDXR = BPW // IW

_mesh = plsc.VectorSubcoreMesh(core_axis_name="c", subcore_axis_name="s")


@functools.partial(
    pl.kernel,
    mesh=_mesh,
    out_type=jax.ShapeDtypeStruct((B, EMB), jnp.float32),
    scratch_types=[
        pltpu.VMEM((IDXR, IW), jnp.int32),
        [pltpu.VMEM((UNIT, EMB), jnp.float32) for _ in range(NBUF)],
        [pltpu.SemaphoreType.DMA for _ in range(NBUF)],
        [pltpu.SemaphoreType.DMA for _ in range(NBUF)],
    ],
    compiler_params=pltpu.CompilerParams(use_tc_tiling_on_sc=False),
)
def _emb_lookup(tokens_hbm, table_hbm, out_hbm, idx_v, bufs, gsems, ssems):
    wid = lax.axis_index("s") * 2 + lax.axis_index("c")
    row0 = wid * IDXR
    base = wid * BPW

    pltpu.sync_copy(tokens_hbm.at[pl.ds(row0, IDXR)], idx_v)

    def issue_gather(u, b):
        for j in range(IPU):
            pltpu.async_copy(
                table_hbm.at[idx_v.at[u * IPU + j]],
                bufs[b].at[pl.ds(j * IW, IW)],
                gsems[b],
            )

    def wait_gather(u, b):
        for j in range(IPU):
            pltpu.make_async_copy(
                table_hbm.at[idx_v.at[u * IPU + j]],
                bufs[b].at[pl.ds(j * IW, IW)],
                gsems[b],
            ).wait()

    def issue_store(u, b):
        pltpu.async_copy(bufs[b], out_hbm.at[pl.ds(base + u * UNIT, UNIT)],
                         ssems[b])

    def wait_store(u, b):
        pltpu.make_async_copy(bufs[b],
                              out_hbm.at[pl.ds(base + u * UNIT, UNIT)],
                              ssems[b]).wait()

    for b in range(NBUF):
        issue_gather(b, b)

    def round_body(r, c):
        for db in range(NBUF):
            u = r * NBUF + db
            bp = (db - 1) % NBUF
            wait_gather(u, db)

            @plsc.parallel_loop(0, UNIT, step=1, unroll=8)
            def _scale(i):
                for jj in range(EMB // 16):
                    sl = pl.ds(jj * 16, 16)
                    bufs[db][i, sl] = bufs[db][i, sl] * SCALE

            @pl.when(u >= 1)
            def _():
                wait_store(u - 1, bp)

            @pl.when((u >= 1) & (u - 1 + NBUF < NU))
            def _():
                issue_gather(u - 1 + NBUF, bp)

            issue_store(u, db)
        return c

    lax.fori_loop(0, NROUND, round_body, 0)
    wait_store(NU - 1, NBUF - 1)


def kernel(tokens, table):
    tokens2d = tokens.astype(jnp.int32).reshape(B // IW, IW)
    out = _emb_lookup(tokens2d, table)
    return out.reshape(tokens.shape[0], tokens.shape[1], EMB)


# speedup vs baseline: 1.8578x; 1.6328x over previous
import functools
import math

import jax
import jax.numpy as jnp
from jax import lax
from jax.experimental import pallas as pl
from jax.experimental.pallas import tpu as pltpu
from jax.experimental.pallas import tpu_sc as plsc

EMB = 64
SCALE = math.sqrt(EMB)

B = 4096 * 200
NW = 32
UNIT = 128
NU_ALL = B // UNIT
NU = NU_ALL // NW
NBUF = 4
NROUND = NU // NBUF
PITCH = 132

_mesh = plsc.VectorSubcoreMesh(core_axis_name="c", subcore_axis_name="s")


@functools.partial(
    pl.kernel,
    mesh=_mesh,
    out_type=jax.ShapeDtypeStruct((200, 8, 32, 8, 128), jnp.float32),
    scratch_types=[
        pltpu.VMEM((NU, UNIT), jnp.int32),
        [pltpu.VMEM((UNIT, EMB), jnp.float32) for _ in range(NBUF)],
        [pltpu.VMEM((EMB, PITCH), jnp.float32) for _ in range(NBUF)],
        [pltpu.SemaphoreType.DMA for _ in range(NBUF)],
        [pltpu.SemaphoreType.DMA for _ in range(NBUF)],
    ],
    compiler_params=pltpu.CompilerParams(
        use_tc_tiling_on_sc=False, needs_layout_passes=False),
)
def _emb_lookup(tokens_hbm, table_hbm, out_hbm, idx_v, gbufs, tbufs,
                gsems, ssems):
    wid = lax.axis_index("s") * 2 + lax.axis_index("c")
    u0 = wid * NU

    pltpu.sync_copy(tokens_hbm.at[pl.ds(u0, NU)], idx_v)

    def issue_gather(lu, b):
        pltpu.async_copy(table_hbm.at[idx_v.at[lu]], gbufs[b], gsems[b])

    def wait_gather(lu, b):
        pltpu.make_async_copy(
            table_hbm.at[idx_v.at[lu]], gbufs[b], gsems[b]).wait()

    def out_tiles(lu, b):
        u = u0 + lu
        i1 = u // 32
        i0g = lax.rem(u, 32)
        return [(tbufs[b].at[pl.ds(8 * jg, 8), pl.ds(0, 128)],
                 out_hbm.at[i1, jg, i0g]) for jg in range(8)]

    def issue_store(lu, b):
        for src, dst in out_tiles(lu, b):
            pltpu.async_copy(src, dst, ssems[b])

    def wait_store(lu, b):
        for src, dst in out_tiles(lu, b):
            pltpu.make_async_copy(src, dst, ssems[b]).wait()

    rows16 = [lax.iota(jnp.int32, 16) + 16 * j for j in range(EMB // 16)]

    for b in range(NBUF):
        issue_gather(b, b)

    def round_body(r, c):
        for db in range(NBUF):
            lu = r * NBUF + db
            bp = (db - 1) % NBUF
            wait_gather(lu, db)

            @plsc.parallel_loop(0, UNIT, step=1, unroll=4)
            def _tr(t):
                col = jnp.full((16,), t, jnp.int32)
                for j in range(EMB // 16):
                    v = gbufs[db][t, pl.ds(16 * j, 16)]
                    plsc.store_scatter(tbufs[db], [rows16[j], col],
                                       v * SCALE)

            @pl.when(lu >= 1)
            def _():
                wait_store(lu - 1, bp)

            @pl.when((lu >= 1) & (lu - 1 + NBUF < NU))
            def _():
                issue_gather(lu - 1 + NBUF, bp)

            issue_store(lu, db)
        return c

    lax.fori_loop(0, NROUND, round_body, 0)
    wait_store(NU - 1, NBUF - 1)


def kernel(tokens, table):
    tokens_u = tokens.T.astype(jnp.int32).reshape(NU_ALL, UNIT)
    o5 = _emb_lookup(tokens_u, table)
    return o5.transpose(2, 4, 0, 1, 3).reshape(4096, 200, EMB)


# speedup vs baseline: 1.8630x; 1.0028x over previous
import functools
import math

import jax
import jax.numpy as jnp
from jax import lax
from jax.experimental import pallas as pl
from jax.experimental.pallas import tpu as pltpu
from jax.experimental.pallas import tpu_sc as plsc

EMB = 64
SCALE = math.sqrt(EMB)

B = 4096 * 200
NW = 32
UNIT = 128
NU_ALL = B // UNIT
NU = NU_ALL // NW
NBUF = 5
NROUND = NU // NBUF
PITCH = 132

_mesh = plsc.VectorSubcoreMesh(core_axis_name="c", subcore_axis_name="s")


@functools.partial(
    pl.kernel,
    mesh=_mesh,
    out_type=jax.ShapeDtypeStruct((200, 8, 32, 8, 128), jnp.float32),
    scratch_types=[
        pltpu.VMEM((NU, UNIT), jnp.int32),
        [pltpu.VMEM((UNIT, EMB), jnp.float32) for _ in range(NBUF)],
        [pltpu.VMEM((EMB, PITCH), jnp.float32) for _ in range(NBUF)],
        [pltpu.SemaphoreType.DMA for _ in range(NBUF)],
        [pltpu.SemaphoreType.DMA for _ in range(NBUF)],
    ],
    compiler_params=pltpu.CompilerParams(
        use_tc_tiling_on_sc=False, needs_layout_passes=False),
)
def _emb_lookup(tokens_hbm, table_hbm, out_hbm, idx_v, gbufs, tbufs,
                gsems, ssems):
    wid = lax.axis_index("s") * 2 + lax.axis_index("c")
    u0 = wid * NU

    pltpu.sync_copy(tokens_hbm.at[pl.ds(u0, NU)], idx_v)

    def issue_gather(lu, b):
        pltpu.async_copy(table_hbm.at[idx_v.at[lu]], gbufs[b], gsems[b])

    def wait_gather(lu, b):
        pltpu.make_async_copy(
            table_hbm.at[idx_v.at[lu]], gbufs[b], gsems[b]).wait()

    def out_tiles(lu, b):
        u = u0 + lu
        i1 = u // 32
        i0g = lax.rem(u, 32)
        return [(tbufs[b].at[pl.ds(8 * jg, 8), pl.ds(0, 128)],
                 out_hbm.at[i1, jg, i0g]) for jg in range(8)]

    def issue_store(lu, b):
        for src, dst in out_tiles(lu, b):
            pltpu.async_copy(src, dst, ssems[b])

    def wait_store(lu, b):
        for src, dst in out_tiles(lu, b):
            pltpu.make_async_copy(src, dst, ssems[b]).wait()

    rows16 = [lax.iota(jnp.int32, 16) + 16 * j for j in range(EMB // 16)]

    for b in range(NBUF):
        issue_gather(b, b)

    def round_body(r, c):
        for db in range(NBUF):
            lu = r * NBUF + db
            bp = (db - 1) % NBUF
            wait_gather(lu, db)

            @plsc.parallel_loop(0, UNIT, step=1, unroll=4)
            def _tr(t):
                col = jnp.full((16,), t, jnp.int32)
                for j in range(EMB // 16):
                    v = gbufs[db][t, pl.ds(16 * j, 16)]
                    plsc.store_scatter(tbufs[db], [rows16[j], col],
                                       v * SCALE)

            @pl.when(lu >= 1)
            def _():
                wait_store(lu - 1, bp)

            @pl.when((lu >= 1) & (lu - 1 + NBUF < NU))
            def _():
                issue_gather(lu - 1 + NBUF, bp)

            issue_store(lu, db)
        return c

    lax.fori_loop(0, NROUND, round_body, 0)
    wait_store(NU - 1, NBUF - 1)


def kernel(tokens, table):
    tokens_u = tokens.T.astype(jnp.int32).reshape(NU_ALL, UNIT)
    o5 = _emb_lookup(tokens_u, table)
    return o5.transpose(2, 4, 0, 1, 3).reshape(4096, 200, EMB)


# speedup vs baseline: 1.8695x; 1.0035x over previous
import functools
import math

import jax
import jax.numpy as jnp
from jax import lax
from jax.experimental import pallas as pl
from jax.experimental.pallas import tpu as pltpu
from jax.experimental.pallas import tpu_sc as plsc

EMB = 64
SCALE = math.sqrt(EMB)

B = 4096 * 200
NW = 32
UNIT = 128
NU_ALL = B // UNIT
NU = NU_ALL // NW
NBUF = 5
NROUND = NU // NBUF
PITCH = 132

_mesh = plsc.VectorSubcoreMesh(core_axis_name="c", subcore_axis_name="s")


@functools.partial(
    pl.kernel,
    mesh=_mesh,
    out_type=jax.ShapeDtypeStruct((200, 8, 32, 8, 128), jnp.float32),
    scratch_types=[
        pltpu.VMEM((NU, UNIT), jnp.int32),
        [pltpu.VMEM((UNIT, EMB), jnp.float32) for _ in range(NBUF)],
        [pltpu.VMEM((8, 8, PITCH), jnp.float32) for _ in range(NBUF)],
        [pltpu.SemaphoreType.DMA for _ in range(NBUF)],
        [pltpu.SemaphoreType.DMA for _ in range(NBUF)],
    ],
    compiler_params=pltpu.CompilerParams(
        use_tc_tiling_on_sc=False, needs_layout_passes=False),
)
def _emb_lookup(tokens_hbm, table_hbm, out_hbm, idx_v, gbufs, tbufs,
                gsems, ssems):
    wid = lax.axis_index("s") * 2 + lax.axis_index("c")
    u0 = wid * NU

    pltpu.sync_copy(tokens_hbm.at[pl.ds(u0, NU)], idx_v)

    def issue_gather(lu, b):
        pltpu.async_copy(table_hbm.at[idx_v.at[lu]], gbufs[b], gsems[b])

    def wait_gather(lu, b):
        pltpu.make_async_copy(
            table_hbm.at[idx_v.at[lu]], gbufs[b], gsems[b]).wait()

    def out_tile(lu, b):
        u = u0 + lu
        i1 = u // 32
        i0g = lax.rem(u, 32)
        return (tbufs[b].at[:, :, pl.ds(0, 128)], out_hbm.at[i1, :, i0g])

    def issue_store(lu, b):
        src, dst = out_tile(lu, b)
        pltpu.async_copy(src, dst, ssems[b])

    def wait_store(lu, b):
        src, dst = out_tile(lu, b)
        pltpu.make_async_copy(src, dst, ssems[b]).wait()

    dims16 = [lax.iota(jnp.int32, 16) + 16 * j for j in range(EMB // 16)]
    jg16 = [d // 8 for d in dims16]
    jl16 = [lax.rem(d, 8) for d in dims16]

    for b in range(NBUF):
        issue_gather(b, b)

    def round_body(r, c):
        for db in range(NBUF):
            lu = r * NBUF + db
            bp = (db - 1) % NBUF
            wait_gather(lu, db)

            @plsc.parallel_loop(0, UNIT, step=1, unroll=4)
            def _tr(t):
                col = jnp.full((16,), t, jnp.int32)
                for j in range(EMB // 16):
                    v = gbufs[db][t, pl.ds(16 * j, 16)]
                    plsc.store_scatter(tbufs[db], [jg16[j], jl16[j], col],
                                       v * SCALE)

            @pl.when(lu >= 1)
            def _():
                wait_store(lu - 1, bp)

            @pl.when((lu >= 1) & (lu - 1 + NBUF < NU))
            def _():
                issue_gather(lu - 1 + NBUF, bp)

            issue_store(lu, db)
        return c

    lax.fori_loop(0, NROUND, round_body, 0)
    wait_store(NU - 1, NBUF - 1)


def kernel(tokens, table):
    tokens_u = tokens.T.astype(jnp.int32).reshape(NU_ALL, UNIT)
    o5 = _emb_lookup(tokens_u, table)
    return o5.transpose(2, 4, 0, 1, 3).reshape(4096, 200, EMB)
